# Initial kernel scaffold; baseline (speedup 1.0000x reference)
import jax
import jax.numpy as jnp
from jax.experimental import pallas as pl


def kernel(x, dists_max, dists_argmax, batch, pre_W, pre_b, c1_dW1, c1_db1, c1_dW2, c1_db2, c1_hW, c1_hb, c1_pW, c1_pb, c2_dW1, c2_db1, c2_dW2, c2_db2, c2_hW, c2_hb, c2_pW, c2_pb, pred_W, pred_b):
    raise NotImplementedError("write your pallas kernel here")


# speedup vs baseline: 1.6525x; 1.6525x over previous
import functools

import jax
import jax.numpy as jnp
from jax import lax
from jax.experimental import pallas as pl
from jax.experimental.pallas import tpu as pltpu
from jax.experimental.pallas import tpu_sc as plsc

N = 10000
K = 32
D = 128
H = 128
G = 16

NC = 2
NS = 16
NW = NC * NS
NPW = 320
NPAD = NW * NPW
CN = 4
NCH = NPW // CN
RPC = CN * K

BD = 1024
BT = 8192


def _dense_pre_body(x_ref, preWT, preb, WlT, WrT, hb, u_ref, s_ref):
    h0 = jnp.dot(x_ref[...], preWT[...], preferred_element_type=jnp.float32)
    h0 = h0 + preb[...]
    u_ref[...] = jnp.dot(h0, WlT[...], preferred_element_type=jnp.float32)
    s_ref[...] = jnp.dot(h0, WrT[...], preferred_element_type=jnp.float32) + hb[...]


def _dense_mid_body(g_ref, WlT, WrT, hb, u_ref, s_ref):
    h1 = jnp.maximum(g_ref[...], 0.0)
    u_ref[...] = jnp.dot(h1, WlT[...], preferred_element_type=jnp.float32)
    s_ref[...] = jnp.dot(h1, WrT[...], preferred_element_type=jnp.float32) + hb[...]


def _dist_mlp_body(t_ref, w1a, b1a, w2a, b2a, w1b, b1b, w2b, b2b, d1_ref, d2_ref):
    t = t_ref[...][:, None]
    qa = jnp.maximum(t * w1a[...] + b1a[...], 0.0)
    d1_ref[...] = jnp.dot(qa, w2a[...], preferred_element_type=jnp.float32)[:, 0] + b2a[0, 0]
    qb = jnp.maximum(t * w1b[...] + b1b[...], 0.0)
    d2_ref[...] = jnp.dot(qb, w2b[...], preferred_element_type=jnp.float32)[:, 0] + b2b[0, 0]


def _pool_body(g_ref, batch_ref, predWc, predb, out_ref):
    p = jnp.dot(g_ref[...], predWc[...], preferred_element_type=jnp.float32)
    b = batch_ref[...]
    oh = (b[None, :] == lax.broadcasted_iota(jnp.int32, (G, NPAD), 0)).astype(jnp.float32)
    sums = jnp.dot(oh, p, preferred_element_type=jnp.float32)
    cnt = jnp.sum(oh, axis=1, keepdims=True)
    out_ref[...] = sums / jnp.maximum(cnt, 1.0) + predb[...]


def _wspec(shape):
    return pl.BlockSpec(shape, lambda i: (0,) * len(shape))


def _dense_pre(xp, preWT, preb, WlT, WrT, hb):
    grid = (NPAD // BD,)
    return pl.pallas_call(
        _dense_pre_body,
        grid=grid,
        in_specs=[
            pl.BlockSpec((BD, D), lambda i: (i, 0)),
            _wspec((D, D)), _wspec((1, D)), _wspec((D, H)), _wspec((D, H)), _wspec((1, H)),
        ],
        out_specs=[pl.BlockSpec((BD, H), lambda i: (i, 0)),
                   pl.BlockSpec((BD, H), lambda i: (i, 0))],
        out_shape=[jax.ShapeDtypeStruct((NPAD, H), jnp.float32),
                   jax.ShapeDtypeStruct((NPAD, H), jnp.float32)],
    )(xp, preWT, preb, WlT, WrT, hb)


def _dense_mid(g1, WlT, WrT, hb):
    grid = (NPAD // BD,)
    return pl.pallas_call(
        _dense_mid_body,
        grid=grid,
        in_specs=[
            pl.BlockSpec((BD, H), lambda i: (i, 0)),
            _wspec((H, H)), _wspec((H, H)), _wspec((1, H)),
        ],
        out_specs=[pl.BlockSpec((BD, H), lambda i: (i, 0)),
                   pl.BlockSpec((BD, H), lambda i: (i, 0))],
        out_shape=[jax.ShapeDtypeStruct((NPAD, H), jnp.float32),
                   jax.ShapeDtypeStruct((NPAD, H), jnp.float32)],
    )(g1, WlT, WrT, hb)


def _dist_mlp(tflat, w1a, b1a, w2a, b2a, w1b, b1b, w2b, b2b):
    M = NPAD * K
    grid = (M // BT,)
    return pl.pallas_call(
        _dist_mlp_body,
        grid=grid,
        in_specs=[
            pl.BlockSpec((BT,), lambda i: (i,)),
            _wspec((1, H)), _wspec((1, H)), _wspec((H, 1)), _wspec((1, 1)),
            _wspec((1, H)), _wspec((1, H)), _wspec((H, 1)), _wspec((1, 1)),
        ],
        out_specs=[pl.BlockSpec((BT,), lambda i: (i,)),
                   pl.BlockSpec((BT,), lambda i: (i,))],
        out_shape=[jax.ShapeDtypeStruct((M,), jnp.float32),
                   jax.ShapeDtypeStruct((M,), jnp.float32)],
    )(tflat, w1a, b1a, w2a, b2a, w1b, b1b, w2b, b2b)


def _pool(g2, bp, predWc, predb):
    return pl.pallas_call(
        _pool_body,
        out_shape=jax.ShapeDtypeStruct((G, 1), jnp.float32),
    )(g2, bp, predWc, predb)


def _make_sc_layer():
    mesh = plsc.VectorSubcoreMesh(core_axis_name="c", subcore_axis_name="s")

    @functools.partial(
        pl.kernel,
        mesh=mesh,
        out_type=jax.ShapeDtypeStruct((NW, NPW, H), jnp.float32),
        scratch_types=[
            pltpu.VMEM((NCH, 128), jnp.int32),
            pltpu.VMEM((NPW, K), jnp.float32),
            pltpu.VMEM((NPW, H), jnp.float32),
            pltpu.VMEM((RPC, H), jnp.float32),
            pltpu.VMEM((CN, H), jnp.float32),
            pltpu.SemaphoreType.DMA,
        ],
    )
    def sc_layer(table, idx, dw, sv, out, idx_v, d_v, s_v, rows_v, out_v, sem):
        wid = lax.axis_index("s") * NC + lax.axis_index("c")
        pltpu.sync_copy(idx.at[wid], idx_v)
        pltpu.sync_copy(dw.at[wid], d_v)
        pltpu.sync_copy(sv.at[wid], s_v)

        def chunk(ci, carry):
            pltpu.async_copy(table.at[idx_v.at[ci]], rows_v, sem).wait()

            def node(i, c2):
                g = ci * CN + i
                svs = [s_v[g, pl.ds(16 * j, 16)] for j in range(8)]
                dvecs = [d_v[g, pl.ds(16 * m, 16)] for m in range(K // 16)]
                accs = [jnp.zeros((16,), jnp.float32) for _ in range(8)]
                for k in range(K):
                    dsc = dvecs[k // 16][k % 16]
                    bb = jnp.full((16,), dsc, jnp.float32)
                    for j in range(8):
                        r = rows_v[i * K + k, pl.ds(16 * j, 16)]
                        accs[j] = accs[j] + jnp.maximum(bb * r + svs[j], 0.0)
                for j in range(8):
                    out_v[i, pl.ds(16 * j, 16)] = accs[j] * (1.0 / K)
                return c2

            lax.fori_loop(0, CN, node, 0)
            pltpu.sync_copy(out_v, out.at[wid, pl.ds(ci * CN, CN)])
            return carry

        lax.fori_loop(0, NCH, chunk, 0)

    return sc_layer


_sc_layer = _make_sc_layer()


def kernel(x, dists_max, dists_argmax, batch, pre_W, pre_b,
           c1_dW1, c1_db1, c1_dW2, c1_db2, c1_hW, c1_hb, c1_pW, c1_pb,
           c2_dW1, c2_db1, c2_dW2, c2_db2, c2_hW, c2_hb, c2_pW, c2_pb,
           pred_W, pred_b):
    pad = NPAD - N
    xp = jnp.pad(x, ((0, pad), (0, 0)))
    tp = jnp.pad(dists_max, ((0, pad), (0, 0)))
    ap = jnp.pad(dists_argmax.astype(jnp.int32), ((0, pad), (0, 0)))
    bp = jnp.pad(batch.astype(jnp.int32), (0, pad), constant_values=G)

    idx = ap.reshape(NW, NCH, 128)

    u1, s1 = _dense_pre(
        xp, pre_W.T, pre_b[None], c1_hW[:, :D].T, c1_hW[:, D:].T, c1_hb[None])

    d1f, d2f = _dist_mlp(
        tp.reshape(-1),
        c1_dW1.T, c1_db1[None], c1_dW2.T, c1_db2[None, :],
        c2_dW1.T, c2_db1[None], c2_dW2.T, c2_db2[None, :])

    g1 = _sc_layer(u1, idx, d1f.reshape(NW, NPW, K), s1.reshape(NW, NPW, H))
    g1 = g1.reshape(NPAD, H)

    u2, s2 = _dense_mid(g1, c2_hW[:, :H].T, c2_hW[:, H:].T, c2_hb[None])

    g2 = _sc_layer(u2, idx, d2f.reshape(NW, NPW, K), s2.reshape(NW, NPW, H))
    g2 = g2.reshape(NPAD, H)

    return _pool(g2, bp, pred_W.T, pred_b[None])
